# Initial kernel scaffold; baseline (speedup 1.0000x reference)
import jax
import jax.numpy as jnp
from jax.experimental import pallas as pl


def kernel(x, nbr_idx, W1, gamma1, beta1, W2, gamma2, beta2):
    raise NotImplementedError("write your pallas kernel here")


# speedup vs baseline: 17.6879x; 17.6879x over previous
import functools
import jax
import jax.numpy as jnp
from jax import lax
from jax.experimental import pallas as pl
from jax.experimental.pallas import tpu as pltpu
from jax.experimental.pallas import tpu_sc as plsc

N = 200000
C = 32
K = 27
EPS = 1e-5

NC = 2
NS = 16
NW = NC * NS

R = 64
GP, GSZ = 18, 96
assert GP * GSZ == R * K
CPW = 98
NPAD = NW * CPW * R
assert NPAD >= N and N % R == 0
VALID_CHUNKS = N // R

MB = 800


def _matmul_body(x_ref, w_ref, o_ref):
    o_ref[...] = jnp.dot(x_ref[...], w_ref[...],
                         preferred_element_type=jnp.float32)


def _tc_matmul(x, w_mat):
    n = x.shape[0]
    kc = w_mat.shape[1]
    return pl.pallas_call(
        _matmul_body,
        grid=(n // MB,),
        in_specs=[
            pl.BlockSpec((MB, C), lambda i: (i, 0)),
            pl.BlockSpec((C, kc), lambda i: (0, 0)),
        ],
        out_specs=pl.BlockSpec((MB, kc), lambda i: (i, 0)),
        out_shape=jax.ShapeDtypeStruct((n, kc), jnp.float32),
    )(x, w_mat)


def _gather_sum_body(table, idx_hbm, out_hbm, stats_hbm,
                     idx_v, buf, acc, stats_v, sem):
    wid = lax.axis_index("s") * NC + lax.axis_index("c")

    def chunk(c, carry):
        s0, s1, q0, q1 = carry
        pltpu.sync_copy(idx_hbm.at[wid, c], idx_v)
        copies = [
            pltpu.async_copy(table.at[idx_v.at[g]],
                             buf.at[pl.ds(g * GSZ, GSZ)], sem)
            for g in range(GP)
        ]
        for cp in copies:
            cp.wait()

        chunk_id = wid * CPW + c
        valid = chunk_id < VALID_CHUNKS

        def site(n, carry2):
            s0, s1, q0, q1 = carry2
            a0 = buf[n * K, pl.ds(0, 16)]
            a1 = buf[n * K, pl.ds(16, 16)]
            for k in range(1, K):
                a0 = a0 + buf[n * K + k, pl.ds(0, 16)]
                a1 = a1 + buf[n * K + k, pl.ds(16, 16)]
            acc[n, pl.ds(0, 16)] = a0
            acc[n, pl.ds(16, 16)] = a1
            b0 = jnp.where(valid, a0, 0.0)
            b1 = jnp.where(valid, a1, 0.0)
            return (s0 + b0, s1 + b1, q0 + b0 * b0, q1 + b1 * b1)

        carry = lax.fori_loop(0, R, site, (s0, s1, q0, q1), unroll=False)
        pltpu.sync_copy(acc, out_hbm.at[pl.ds(chunk_id * R, R)])
        return carry

    z = jnp.zeros((16,), jnp.float32)
    s0, s1, q0, q1 = lax.fori_loop(0, CPW, chunk, (z, z, z, z), unroll=False)
    stats_v[pl.ds(0, 16)] = s0
    stats_v[pl.ds(16, 16)] = s1
    stats_v[pl.ds(32, 16)] = q0
    stats_v[pl.ds(48, 16)] = q1
    pltpu.sync_copy(stats_v, stats_hbm.at[wid])


def _sc_gather_sum(table_flat, idx4):
    mesh = plsc.VectorSubcoreMesh(core_axis_name="c", subcore_axis_name="s",
                                  num_cores=NC, num_subcores=NS)
    out_flat, stats_flat = pl.kernel(
        _gather_sum_body,
        out_type=[
            jax.ShapeDtypeStruct((NPAD, C), jnp.float32),
            jax.ShapeDtypeStruct((NW, 2 * C), jnp.float32),
        ],
        mesh=mesh,
        scratch_types=[
            pltpu.VMEM((GP, GSZ), jnp.int32),
            pltpu.VMEM((R * K, C), jnp.float32),
            pltpu.VMEM((R, C), jnp.float32),
            pltpu.VMEM((2 * C,), jnp.float32),
            pltpu.SemaphoreType.DMA,
        ],
        compiler_params=pltpu.CompilerParams(use_tc_tiling_on_sc=False),
    )(table_flat, idx4)
    return out_flat, stats_flat


def _bn_scale_shift(stats_ref, g_ref, b_ref):
    s = jnp.sum(stats_ref[...], axis=0)
    mean = s[:C] * (1.0 / N)
    var = s[C:] * (1.0 / N) - mean * mean
    scale = g_ref[...] * lax.rsqrt(var + EPS)
    shift = b_ref[...] - mean * scale
    return scale, shift


def _bn_relu_matmul_body(h_ref, stats_ref, g_ref, b_ref, w_ref, o_ref):
    scale, shift = _bn_scale_shift(stats_ref, g_ref, b_ref)
    z = jnp.maximum(h_ref[...] * scale[None, :] + shift[None, :], 0.0)
    o_ref[...] = jnp.dot(z, w_ref[...], preferred_element_type=jnp.float32)


def _tc_bn_relu_matmul(h, stats, gamma, beta, w_mat):
    kc = w_mat.shape[1]
    return pl.pallas_call(
        _bn_relu_matmul_body,
        grid=(N // MB,),
        in_specs=[
            pl.BlockSpec((MB, C), lambda i: (i, 0)),
            pl.BlockSpec((NW, 2 * C), lambda i: (0, 0)),
            pl.BlockSpec((C,), lambda i: (0,)),
            pl.BlockSpec((C,), lambda i: (0,)),
            pl.BlockSpec((C, kc), lambda i: (0, 0)),
        ],
        out_specs=pl.BlockSpec((MB, kc), lambda i: (i, 0)),
        out_shape=jax.ShapeDtypeStruct((N, kc), jnp.float32),
    )(h, stats, gamma, beta, w_mat)


def _bn_res_relu_body(h_ref, stats_ref, g_ref, b_ref, x_ref, o_ref):
    scale, shift = _bn_scale_shift(stats_ref, g_ref, b_ref)
    o_ref[...] = jnp.maximum(
        h_ref[...] * scale[None, :] + shift[None, :] + x_ref[...], 0.0)


def _tc_bn_res_relu(h, stats, gamma, beta, x):
    return pl.pallas_call(
        _bn_res_relu_body,
        grid=(N // MB,),
        in_specs=[
            pl.BlockSpec((MB, C), lambda i: (i, 0)),
            pl.BlockSpec((NW, 2 * C), lambda i: (0, 0)),
            pl.BlockSpec((C,), lambda i: (0,)),
            pl.BlockSpec((C,), lambda i: (0,)),
            pl.BlockSpec((MB, C), lambda i: (i, 0)),
        ],
        out_specs=pl.BlockSpec((MB, C), lambda i: (i, 0)),
        out_shape=jax.ShapeDtypeStruct((N, C), jnp.float32),
    )(h, stats, gamma, beta, x)


@jax.jit
def kernel(x, nbr_idx, W1, gamma1, beta1, W2, gamma2, beta2):
    w1m = W1.transpose(1, 0, 2).reshape(C, K * C)
    w2m = W2.transpose(1, 0, 2).reshape(C, K * C)

    fi = nbr_idx * K + jnp.arange(K, dtype=jnp.int32)[None, :]
    fi = jnp.pad(fi, ((0, NPAD - N), (0, 0)))
    idx4 = fi.reshape(NW, CPW, GP, GSZ)

    y1 = _tc_matmul(x, w1m)
    h1, st1 = _sc_gather_sum(y1.reshape(N * K, C), idx4)
    y2 = _tc_bn_relu_matmul(h1, st1, gamma1, beta1, w2m)
    h2, st2 = _sc_gather_sum(y2.reshape(N * K, C), idx4)
    return _tc_bn_res_relu(h2, st2, gamma2, beta2, x)
